# baseline (speedup 1.0000x reference)
import functools

import jax
import jax.numpy as jnp
from jax import lax
from jax.experimental import pallas as pl
from jax.experimental.pallas import tpu as pltpu
from jax.experimental.pallas import tpu_sc as plsc

DIM = 300
DIM_PAD = 304
CHUNK = 128


def _embed_gather(idx_grp, table, n_rows, num_cores, num_subcores):
    n_chunks = idx_grp.shape[1]
    n_per_w = n_chunks * CHUNK

    mesh = plsc.VectorSubcoreMesh(core_axis_name="c", subcore_axis_name="s")

    @functools.partial(
        pl.kernel,
        mesh=mesh,
        compiler_params=pltpu.CompilerParams(use_tc_tiling_on_sc=False),
        out_type=jax.ShapeDtypeStruct((n_rows, DIM_PAD), jnp.float32),
        scratch_types=[
            pltpu.VMEM((n_chunks, CHUNK), jnp.int32),
            pltpu.VMEM((CHUNK, DIM_PAD), jnp.float32),
            pltpu.SemaphoreType.DMA,
        ],
    )
    def k(idx_hbm, table_hbm, out_hbm, idx_v, rows, gsem):
        wid = lax.axis_index("s") * num_cores + lax.axis_index("c")
        base = wid * n_per_w
        pltpu.sync_copy(idx_hbm.at[wid], idx_v)

        def body(j, carry):
            pltpu.async_copy(table_hbm.at[idx_v.at[j]], rows, gsem).wait()
            pltpu.sync_copy(rows, out_hbm.at[pl.ds(base + j * CHUNK, CHUNK)])
            return carry

        lax.fori_loop(0, n_chunks, body, 0)

    return k(idx_grp, table)


def kernel(xc, xw, table):
    del xc
    b, l = xw.shape
    n = b * l
    info = plsc.get_sparse_core_info()
    nw = info.num_cores * info.num_subcores
    n_chunks = n // (nw * CHUNK)
    idx = xw.reshape(nw, n_chunks, CHUNK).astype(jnp.int32)
    table_p = jnp.pad(table, ((0, 0), (0, DIM_PAD - DIM)))
    out = _embed_gather(idx, table_p, n, info.num_cores, info.num_subcores)
    return out[:, :DIM].reshape(b, l, DIM)


# speedup vs baseline: 1.3564x; 1.3564x over previous
import functools

import jax
import jax.numpy as jnp
from jax import lax
from jax.experimental import pallas as pl
from jax.experimental.pallas import tpu as pltpu
from jax.experimental.pallas import tpu_sc as plsc

DIM = 300
DIM_PAD = 384
CHUNK = 128


def _embed_gather(idx_grp, table, n_rows, num_cores, num_subcores):
    n_chunks = idx_grp.shape[1]
    n_per_w = n_chunks * CHUNK

    mesh = plsc.VectorSubcoreMesh(core_axis_name="c", subcore_axis_name="s")

    @functools.partial(
        pl.kernel,
        mesh=mesh,
        out_type=jax.ShapeDtypeStruct((n_rows, DIM_PAD), jnp.float32),
        scratch_types=[
            pltpu.VMEM((n_chunks, CHUNK), jnp.int32),
            pltpu.VMEM((CHUNK, DIM_PAD), jnp.float32),
            pltpu.SemaphoreType.DMA,
        ],
    )
    def k(idx_hbm, table_hbm, out_hbm, idx_v, rows, gsem):
        wid = lax.axis_index("s") * num_cores + lax.axis_index("c")
        base = wid * n_per_w
        pltpu.sync_copy(idx_hbm.at[wid], idx_v)

        def body(j, carry):
            pltpu.async_copy(table_hbm.at[idx_v.at[j]], rows, gsem).wait()
            pltpu.sync_copy(rows, out_hbm.at[pl.ds(base + j * CHUNK, CHUNK)])
            return carry

        lax.fori_loop(0, n_chunks, body, 0)

    return k(idx_grp, table)


def kernel(xc, xw, table):
    del xc
    b, l = xw.shape
    n = b * l
    info = plsc.get_sparse_core_info()
    nw = info.num_cores * info.num_subcores
    n_chunks = n // (nw * CHUNK)
    idx = xw.reshape(nw, n_chunks, CHUNK).astype(jnp.int32)
    table_p = jnp.pad(table, ((0, 0), (0, DIM_PAD - DIM)))
    out = _embed_gather(idx, table_p, n, info.num_cores, info.num_subcores)
    return out[:, :DIM].reshape(b, l, DIM)


# speedup vs baseline: 1.8540x; 1.3668x over previous
import functools

import jax
import jax.numpy as jnp
from jax import lax
from jax.experimental import pallas as pl
from jax.experimental.pallas import tpu as pltpu
from jax.experimental.pallas import tpu_sc as plsc

DIM = 300
DIM_PAD = 384
CHUNK = 128


def _embed_gather(idx_grp, table, n_rows, num_cores, num_subcores):
    n_chunks = idx_grp.shape[1]
    n_per_w = n_chunks * CHUNK

    mesh = plsc.VectorSubcoreMesh(core_axis_name="c", subcore_axis_name="s")

    @functools.partial(
        pl.kernel,
        mesh=mesh,
        out_type=jax.ShapeDtypeStruct((n_rows, DIM_PAD), jnp.float32),
        scratch_types=[
            pltpu.VMEM((n_chunks, CHUNK), jnp.int32),
            pltpu.VMEM((CHUNK, DIM_PAD), jnp.float32),
            pltpu.SemaphoreType.DMA,
        ],
    )
    def k(idx_hbm, table_hbm, out_hbm, idx_v, rows, gsem):
        wid = lax.axis_index("s") * num_cores + lax.axis_index("c")
        base = wid * n_per_w
        pltpu.sync_copy(idx_hbm.at[wid], idx_v)

        def body(j, carry):
            pltpu.async_copy(table_hbm.at[idx_v.at[j]], rows, gsem).wait()
            pltpu.sync_copy(rows, out_hbm.at[pl.ds(base + j * CHUNK, CHUNK)])
            return carry

        lax.fori_loop(0, n_chunks, body, 0)

    return k(idx_grp, table)


def _pad_cols_tc(table):
    v = table.shape[0]
    blk = 2000

    def body(in_ref, out_ref):
        out_ref[:, :DIM] = in_ref[...]

    return pl.pallas_call(
        body,
        grid=(v // blk,),
        in_specs=[pl.BlockSpec((blk, DIM), lambda i: (i, 0))],
        out_specs=pl.BlockSpec((blk, DIM_PAD), lambda i: (i, 0)),
        out_shape=jax.ShapeDtypeStruct((v, DIM_PAD), jnp.float32),
    )(table)


def kernel(xc, xw, table):
    del xc
    b, l = xw.shape
    n = b * l
    info = plsc.get_sparse_core_info()
    nw = info.num_cores * info.num_subcores
    n_chunks = n // (nw * CHUNK)
    idx = xw.reshape(nw, n_chunks, CHUNK).astype(jnp.int32)
    table_p = _pad_cols_tc(table)
    out = _embed_gather(idx, table_p, n, info.num_cores, info.num_subcores)
    return out[:, :DIM].reshape(b, l, DIM)


# speedup vs baseline: 2.1251x; 1.1462x over previous
import functools

import jax
import jax.numpy as jnp
from jax import lax
from jax.experimental import pallas as pl
from jax.experimental.pallas import tpu as pltpu
from jax.experimental.pallas import tpu_sc as plsc

DIM = 300
DIM_PAD = 384
SEQ = 50
SEQ_PAD = 56


def _pad_cols_tc(table):
    v = table.shape[0]
    blk = 2000

    def body(in_ref, out_ref):
        out_ref[:, :DIM] = in_ref[...]

    return pl.pallas_call(
        body,
        grid=(v // blk,),
        in_specs=[pl.BlockSpec((blk, DIM), lambda i: (i, 0))],
        out_specs=pl.BlockSpec((blk, DIM_PAD), lambda i: (i, 0)),
        out_shape=jax.ShapeDtypeStruct((v, DIM_PAD), jnp.float32),
    )(table)


def _embed_gather(idx_grp, table, n_batch, num_cores, num_subcores):
    b_per_w = idx_grp.shape[1]

    mesh = plsc.VectorSubcoreMesh(core_axis_name="c", subcore_axis_name="s")

    @functools.partial(
        pl.kernel,
        mesh=mesh,
        out_type=jax.ShapeDtypeStruct((n_batch * SEQ_PAD, DIM_PAD), jnp.float32),
        scratch_types=[
            pltpu.VMEM((b_per_w, SEQ_PAD), jnp.int32),
            pltpu.VMEM((SEQ_PAD, DIM_PAD), jnp.float32),
            pltpu.SemaphoreType.DMA,
        ],
    )
    def k(idx_hbm, table_hbm, out_hbm, idx_v, rows, gsem):
        wid = lax.axis_index("s") * num_cores + lax.axis_index("c")
        base_b = wid * b_per_w
        pltpu.sync_copy(idx_hbm.at[wid], idx_v)

        def body(jb, carry):
            pltpu.async_copy(table_hbm.at[idx_v.at[jb]], rows, gsem).wait()
            pltpu.sync_copy(
                rows, out_hbm.at[pl.ds((base_b + jb) * SEQ_PAD, SEQ_PAD)]
            )
            return carry

        lax.fori_loop(0, b_per_w, body, 0)

    return k(idx_grp, table)


def kernel(xc, xw, table):
    del xc
    b, l = xw.shape
    info = plsc.get_sparse_core_info()
    nw = info.num_cores * info.num_subcores
    idx = xw.reshape(nw, b // nw, l).astype(jnp.int32)
    idx = jnp.pad(idx, ((0, 0), (0, 0), (0, SEQ_PAD - SEQ)), mode="edge")
    table_p = _pad_cols_tc(table)
    out = _embed_gather(idx, table_p, b, info.num_cores, info.num_subcores)
    return out.reshape(b, SEQ_PAD, DIM_PAD)[:, :SEQ, :DIM]


# speedup vs baseline: 2.5958x; 1.2215x over previous
import functools

import jax
import jax.numpy as jnp
from jax import lax
from jax.experimental import pallas as pl
from jax.experimental.pallas import tpu as pltpu
from jax.experimental.pallas import tpu_sc as plsc

DIM = 300
DIM_PAD = 384
SEQ = 50
SEQ_PAD = 56


def _pad_cols_tc(table):
    v = table.shape[0]
    blk = 2000

    def body(in_ref, out_ref):
        out_ref[:, :DIM] = in_ref[...]

    return pl.pallas_call(
        body,
        grid=(v // blk,),
        in_specs=[pl.BlockSpec((blk, DIM), lambda i: (i, 0))],
        out_specs=pl.BlockSpec((blk, DIM_PAD), lambda i: (i, 0)),
        out_shape=jax.ShapeDtypeStruct((v, DIM_PAD), jnp.float32),
    )(table)


def _embed_gather(idx_grp, table, n_batch, num_cores, num_subcores):
    b_per_w = idx_grp.shape[1]

    mesh = plsc.VectorSubcoreMesh(core_axis_name="c", subcore_axis_name="s")

    nbuf = 4

    @functools.partial(
        pl.kernel,
        mesh=mesh,
        out_type=jax.ShapeDtypeStruct((n_batch * SEQ_PAD, DIM_PAD), jnp.float32),
        scratch_types=[
            pltpu.VMEM((b_per_w, SEQ_PAD), jnp.int32),
            [pltpu.VMEM((SEQ_PAD, DIM_PAD), jnp.float32) for _ in range(nbuf)],
            [pltpu.SemaphoreType.DMA for _ in range(nbuf)],
            [pltpu.SemaphoreType.DMA for _ in range(nbuf)],
        ],
    )
    def k(idx_hbm, table_hbm, out_hbm, idx_v, rows, gsems, wsems):
        wid = lax.axis_index("s") * num_cores + lax.axis_index("c")
        base_b = wid * b_per_w
        pltpu.sync_copy(idx_hbm.at[wid], idx_v)

        def g_start(jb, p):
            pltpu.async_copy(table_hbm.at[idx_v.at[jb]], rows[p], gsems[p])

        def wb_start(jb, p):
            pltpu.async_copy(
                rows[p], out_hbm.at[pl.ds((base_b + jb) * SEQ_PAD, SEQ_PAD)],
                wsems[p],
            )

        for p in range(nbuf):
            g_start(p, p)

        @pl.loop(0, b_per_w, step=nbuf)
        def _(jb):
            for p in range(nbuf):
                j = jb + p
                pltpu.make_async_copy(
                    table_hbm.at[idx_v.at[j]], rows[p], gsems[p]
                ).wait()
                wb_start(j, p)

                @pl.when(j + nbuf < b_per_w)
                def _():
                    pltpu.make_async_copy(
                        rows[p],
                        out_hbm.at[pl.ds((base_b + j) * SEQ_PAD, SEQ_PAD)],
                        wsems[p],
                    ).wait()
                    g_start(j + nbuf, p)

        for p in range(nbuf):
            j = b_per_w - nbuf + p
            pltpu.make_async_copy(
                rows[p],
                out_hbm.at[pl.ds((base_b + j) * SEQ_PAD, SEQ_PAD)],
                wsems[p],
            ).wait()

    return k(idx_grp, table)


def kernel(xc, xw, table):
    del xc
    b, l = xw.shape
    info = plsc.get_sparse_core_info()
    nw = info.num_cores * info.num_subcores
    idx = xw.reshape(nw, b // nw, l).astype(jnp.int32)
    idx = jnp.pad(idx, ((0, 0), (0, 0), (0, SEQ_PAD - SEQ)), mode="edge")
    table_p = _pad_cols_tc(table)
    out = _embed_gather(idx, table_p, b, info.num_cores, info.num_subcores)
    return out.reshape(b, SEQ_PAD, DIM_PAD)[:, :SEQ, :DIM]
